# baseline (speedup 1.0000x reference)
import functools

import jax
import jax.numpy as jnp
from jax import lax
from jax.experimental import pallas as pl
from jax.experimental.pallas import tpu as pltpu
from jax.experimental.pallas import tpu_sc as plsc

D = 768
H = 4 * D
E = 64
S = 4096
NPAIR = 2 * S
B = 256
N_TILES = NPAIR // B + E
P = N_TILES * B

NW = 32
CH = 64


def _router_body(x_ref, wt_ref, logits_ref, sel_ref, rw_ref):
    xb = x_ref[...]
    logits = jnp.dot(xb, wt_ref[...], preferred_element_type=jnp.float32)
    logits_ref[...] = logits
    m = jnp.max(logits, axis=1, keepdims=True)
    p = jnp.exp(logits - m)
    probs = p / jnp.sum(p, axis=1, keepdims=True)
    iota = lax.broadcasted_iota(jnp.int32, probs.shape, 1)
    p1 = jnp.max(probs, axis=1, keepdims=True)
    i1 = jnp.min(jnp.where(probs == p1, iota, E), axis=1, keepdims=True)
    probs2 = jnp.where(iota == i1, -jnp.inf, probs)
    p2 = jnp.max(probs2, axis=1, keepdims=True)
    i2 = jnp.min(jnp.where(probs2 == p2, iota, E), axis=1, keepdims=True)
    s = p1 + p2
    sel_ref[:, 0:1] = i1
    sel_ref[:, 1:2] = i2
    rw_ref[:, 0:1] = p1 / s
    rw_ref[:, 1:2] = p2 / s


def _router(xf, wt):
    return pl.pallas_call(
        _router_body,
        out_shape=(
            jax.ShapeDtypeStruct((S, E), jnp.float32),
            jax.ShapeDtypeStruct((S, 2), jnp.int32),
            jax.ShapeDtypeStruct((S, 2), jnp.float32),
        ),
    )(xf, wt)


def _wid():
    return lax.axis_index("s") * 2 + lax.axis_index("c")


_RPW = P // NW
_NCH = _RPW // CH


def _sc_gather_body(x_hbm, idx_hbm, out_hbm, idx_v, b0, b1, g0, g1, w0, w1):
    base = _wid() * _RPW
    pltpu.sync_copy(idx_hbm.at[pl.ds(base, _RPW)], idx_v)
    bufs, gsem, wsem = (b0, b1), (g0, g1), (w0, w1)
    gcp = [None, None]
    wcp = [None, None]
    gcp[0] = pltpu.async_copy(x_hbm.at[idx_v.at[pl.ds(0, CH)]], bufs[0], gsem[0])
    for i in range(_NCH):
        b = i % 2
        if i + 1 < _NCH:
            nb = (i + 1) % 2
            if wcp[nb] is not None:
                wcp[nb].wait()
            gcp[nb] = pltpu.async_copy(
                x_hbm.at[idx_v.at[pl.ds((i + 1) * CH, CH)]], bufs[nb], gsem[nb])
        gcp[b].wait()
        wcp[b] = pltpu.async_copy(
            bufs[b], out_hbm.at[pl.ds(base + i * CH, CH)], wsem[b])
    wcp[(_NCH - 1) % 2].wait()
    if _NCH > 1:
        wcp[_NCH % 2].wait()


@functools.cache
def _sc_gather():
    return pl.kernel(
        _sc_gather_body,
        out_type=jax.ShapeDtypeStruct((P, D), jnp.float32),
        mesh=plsc.VectorSubcoreMesh(core_axis_name="c", subcore_axis_name="s"),
        scratch_types=[
            pltpu.VMEM((_RPW,), jnp.int32),
            pltpu.VMEM((CH, D), jnp.float32),
            pltpu.VMEM((CH, D), jnp.float32),
            pltpu.SemaphoreType.DMA,
            pltpu.SemaphoreType.DMA,
            pltpu.SemaphoreType.DMA,
            pltpu.SemaphoreType.DMA,
        ],
    )


def _sc_combine_body(ys_hbm, p0_hbm, p1_hbm, out_hbm, i0_v, i1_v, g0_v, g1_v, s0, s1):
    base = _wid() * (S // NW)

    def chunk(ci, carry):
        off = base + ci * CH
        pltpu.sync_copy(p0_hbm.at[pl.ds(off, CH)], i0_v)
        pltpu.sync_copy(p1_hbm.at[pl.ds(off, CH)], i1_v)
        c0 = pltpu.async_copy(ys_hbm.at[i0_v], g0_v, s0)
        c1 = pltpu.async_copy(ys_hbm.at[i1_v], g1_v, s1)
        c0.wait()
        c1.wait()

        def add_one(j, c):
            r = j // (D // 16)
            col = (j % (D // 16)) * 16
            g0_v[r, pl.ds(col, 16)] = g0_v[r, pl.ds(col, 16)] + g1_v[r, pl.ds(col, 16)]
            return c

        lax.fori_loop(0, CH * (D // 16), add_one, 0)
        pltpu.sync_copy(g0_v, out_hbm.at[pl.ds(off, CH)])
        return carry

    lax.fori_loop(0, S // NW // CH, chunk, 0)


@functools.cache
def _sc_combine():
    return pl.kernel(
        _sc_combine_body,
        out_type=jax.ShapeDtypeStruct((S, D), jnp.float32),
        mesh=plsc.VectorSubcoreMesh(core_axis_name="c", subcore_axis_name="s"),
        scratch_types=[
            pltpu.VMEM((CH,), jnp.int32),
            pltpu.VMEM((CH,), jnp.int32),
            pltpu.VMEM((CH, D), jnp.float32),
            pltpu.VMEM((CH, D), jnp.float32),
            pltpu.SemaphoreType.DMA,
            pltpu.SemaphoreType.DMA,
        ],
    )


def _gelu(h):
    return 0.5 * h * (1.0 + lax.erf(h * (2.0 ** -0.5)))


def _gemm_body(te_ref, tv_ref, xs_ref, fc_ref, pj_ref, w_ref, ys_ref):
    @pl.when(tv_ref[pl.program_id(0)] == 1)
    def _():
        xb = xs_ref[...]
        h = lax.dot_general(xb, fc_ref[0], (((1,), (1,)), ((), ())),
                            preferred_element_type=jnp.float32)
        h = _gelu(h)
        y = lax.dot_general(h, pj_ref[0], (((1,), (1,)), ((), ())),
                            preferred_element_type=jnp.float32)
        ys_ref[...] = y * w_ref[...]


def _gemm(tile_expert, tile_valid, xs, fc_w, proj_w, wrow):
    grid_spec = pltpu.PrefetchScalarGridSpec(
        num_scalar_prefetch=2,
        grid=(N_TILES,),
        in_specs=[
            pl.BlockSpec((B, D), lambda i, te, tv: (i, 0)),
            pl.BlockSpec((1, H, D), lambda i, te, tv: (te[i], 0, 0)),
            pl.BlockSpec((1, D, H), lambda i, te, tv: (te[i], 0, 0)),
            pl.BlockSpec((B, 1), lambda i, te, tv: (i, 0)),
        ],
        out_specs=pl.BlockSpec((B, D), lambda i, te, tv: (i, 0)),
    )
    return pl.pallas_call(
        _gemm_body,
        grid_spec=grid_spec,
        out_shape=jax.ShapeDtypeStruct((P, D), jnp.float32),
        compiler_params=pltpu.CompilerParams(
            dimension_semantics=("arbitrary",)),
    )(tile_expert, tile_valid, xs, fc_w, proj_w, wrow)


def kernel(x, router_w, fc_w, proj_w):
    b, s, d = x.shape
    xf = x.reshape(s, d)

    logits, sel, rw = _router(xf, router_w.T)

    e = sel.reshape(-1)
    onehot = (e[:, None] == jnp.arange(E, dtype=e.dtype)).astype(jnp.int16)
    cinc = jnp.cumsum(onehot, axis=0)
    counts = cinc[-1].astype(jnp.int32)
    rank = jnp.take_along_axis(cinc, e[:, None], axis=1)[:, 0].astype(jnp.int32) - 1
    tiles_per = (counts + B - 1) // B
    tile_bound = jnp.cumsum(tiles_per)
    row_start = jnp.concatenate(
        [jnp.zeros((1,), jnp.int32), tile_bound[:-1].astype(jnp.int32)]) * B
    pos = row_start[e] + rank

    tok = jnp.arange(NPAIR, dtype=jnp.int32) // 2
    base_tw = jnp.stack(
        [(jnp.arange(P, dtype=jnp.float32) % S), jnp.zeros((P,), jnp.float32)],
        axis=1)
    upd = jnp.stack([tok.astype(jnp.float32), rw.reshape(-1)], axis=1)
    tw = base_tw.at[pos].set(upd)
    gtok = tw[:, 0].astype(jnp.int32)
    wrow = tw[:, 1:2]
    tile_ids = jnp.arange(N_TILES, dtype=jnp.int32)
    tile_expert = jnp.minimum(
        jnp.searchsorted(tile_bound, tile_ids, side="right"), E - 1
    ).astype(jnp.int32)
    tile_valid = (tile_ids < tile_bound[-1]).astype(jnp.int32)

    xs = _sc_gather()(xf, gtok)
    ys = _gemm(tile_expert, tile_valid, xs, fc_w, proj_w, wrow)

    posm = pos.reshape(s, 2)
    out = _sc_combine()(ys, posm[:, 0], posm[:, 1])
    return out.reshape(b, s, d), logits


# speedup vs baseline: 1.0514x; 1.0514x over previous
import functools

import jax
import jax.numpy as jnp
from jax import lax
from jax.experimental import pallas as pl
from jax.experimental.pallas import tpu as pltpu
from jax.experimental.pallas import tpu_sc as plsc

D = 768
H = 4 * D
E = 64
S = 4096
NPAIR = 2 * S
B = 256
N_TILES = NPAIR // B + E
P = N_TILES * B

NW = 32
CH = 64


def _router_body(x_ref, wt_ref, logits_ref, sel_ref, rw_ref):
    xb = x_ref[...]
    logits = jnp.dot(xb, wt_ref[...], preferred_element_type=jnp.float32)
    logits_ref[...] = logits
    m = jnp.max(logits, axis=1, keepdims=True)
    p = jnp.exp(logits - m)
    probs = p / jnp.sum(p, axis=1, keepdims=True)
    iota = lax.broadcasted_iota(jnp.int32, probs.shape, 1)
    p1 = jnp.max(probs, axis=1, keepdims=True)
    i1 = jnp.min(jnp.where(probs == p1, iota, E), axis=1, keepdims=True)
    probs2 = jnp.where(iota == i1, -jnp.inf, probs)
    p2 = jnp.max(probs2, axis=1, keepdims=True)
    i2 = jnp.min(jnp.where(probs2 == p2, iota, E), axis=1, keepdims=True)
    s = p1 + p2
    sel_ref[:, 0:1] = i1
    sel_ref[:, 1:2] = i2
    rw_ref[:, 0:1] = p1 / s
    rw_ref[:, 1:2] = p2 / s


def _router(xf, wt):
    return pl.pallas_call(
        _router_body,
        out_shape=(
            jax.ShapeDtypeStruct((S, E), jnp.float32),
            jax.ShapeDtypeStruct((S, 2), jnp.int32),
            jax.ShapeDtypeStruct((S, 2), jnp.float32),
        ),
    )(xf, wt)


def _wid():
    return lax.axis_index("s") * 2 + lax.axis_index("c")


_RPW = P // NW
_NCH = _RPW // CH


def _sc_gather_body(x_hbm, idx_hbm, out_hbm, idx_v, b0, b1, g0, g1, w0, w1):
    base = _wid() * _RPW
    pltpu.sync_copy(idx_hbm.at[pl.ds(base, _RPW)], idx_v)
    bufs, gsem, wsem = (b0, b1), (g0, g1), (w0, w1)
    gcp = [None, None]
    wcp = [None, None]
    gcp[0] = pltpu.async_copy(x_hbm.at[idx_v.at[pl.ds(0, CH)]], bufs[0], gsem[0])
    for i in range(_NCH):
        b = i % 2
        if i + 1 < _NCH:
            nb = (i + 1) % 2
            if wcp[nb] is not None:
                wcp[nb].wait()
            gcp[nb] = pltpu.async_copy(
                x_hbm.at[idx_v.at[pl.ds((i + 1) * CH, CH)]], bufs[nb], gsem[nb])
        gcp[b].wait()
        wcp[b] = pltpu.async_copy(
            bufs[b], out_hbm.at[pl.ds(base + i * CH, CH)], wsem[b])
    wcp[(_NCH - 1) % 2].wait()
    if _NCH > 1:
        wcp[_NCH % 2].wait()


@functools.cache
def _sc_gather():
    return pl.kernel(
        _sc_gather_body,
        out_type=jax.ShapeDtypeStruct((P, D), jnp.float32),
        mesh=plsc.VectorSubcoreMesh(core_axis_name="c", subcore_axis_name="s"),
        scratch_types=[
            pltpu.VMEM((_RPW,), jnp.int32),
            pltpu.VMEM((CH, D), jnp.float32),
            pltpu.VMEM((CH, D), jnp.float32),
            pltpu.SemaphoreType.DMA,
            pltpu.SemaphoreType.DMA,
            pltpu.SemaphoreType.DMA,
            pltpu.SemaphoreType.DMA,
        ],
    )


def _sc_combine_body(ys_hbm, p0_hbm, p1_hbm, out_hbm, i0_v, i1_v, g0_v, g1_v, s0, s1):
    base = _wid() * (S // NW)

    def chunk(ci, carry):
        off = base + ci * CH
        pltpu.sync_copy(p0_hbm.at[pl.ds(off, CH)], i0_v)
        pltpu.sync_copy(p1_hbm.at[pl.ds(off, CH)], i1_v)
        c0 = pltpu.async_copy(ys_hbm.at[i0_v], g0_v, s0)
        c1 = pltpu.async_copy(ys_hbm.at[i1_v], g1_v, s1)
        c0.wait()
        c1.wait()

        def add_one(j, c):
            r = j // (D // 16)
            col = (j % (D // 16)) * 16
            g0_v[r, pl.ds(col, 16)] = g0_v[r, pl.ds(col, 16)] + g1_v[r, pl.ds(col, 16)]
            return c

        lax.fori_loop(0, CH * (D // 16), add_one, 0)
        pltpu.sync_copy(g0_v, out_hbm.at[pl.ds(off, CH)])
        return carry

    lax.fori_loop(0, S // NW // CH, chunk, 0)


@functools.cache
def _sc_combine():
    return pl.kernel(
        _sc_combine_body,
        out_type=jax.ShapeDtypeStruct((S, D), jnp.float32),
        mesh=plsc.VectorSubcoreMesh(core_axis_name="c", subcore_axis_name="s"),
        scratch_types=[
            pltpu.VMEM((CH,), jnp.int32),
            pltpu.VMEM((CH,), jnp.int32),
            pltpu.VMEM((CH, D), jnp.float32),
            pltpu.VMEM((CH, D), jnp.float32),
            pltpu.SemaphoreType.DMA,
            pltpu.SemaphoreType.DMA,
        ],
    )


def _gelu(h):
    return 0.5 * h * (1.0 + lax.erf(h * (2.0 ** -0.5)))


def _gemm_body(te_ref, tv_ref, xs_ref, fc_ref, pj_ref, w_ref, ys_ref):
    @pl.when(tv_ref[pl.program_id(0)] == 1)
    def _():
        xb = xs_ref[...]
        h = lax.dot_general(xb, fc_ref[0], (((1,), (1,)), ((), ())),
                            preferred_element_type=jnp.float32)
        h = _gelu(h)
        y = lax.dot_general(h, pj_ref[0], (((1,), (1,)), ((), ())),
                            preferred_element_type=jnp.float32)
        ys_ref[...] = y * w_ref[...]


def _gemm(tile_expert, tile_valid, xs, fc_w, proj_w, wrow):
    grid_spec = pltpu.PrefetchScalarGridSpec(
        num_scalar_prefetch=2,
        grid=(N_TILES,),
        in_specs=[
            pl.BlockSpec((B, D), lambda i, te, tv: (i, 0)),
            pl.BlockSpec((1, H, D), lambda i, te, tv: (te[i], 0, 0)),
            pl.BlockSpec((1, D, H), lambda i, te, tv: (te[i], 0, 0)),
            pl.BlockSpec((B, 1), lambda i, te, tv: (i, 0)),
        ],
        out_specs=pl.BlockSpec((B, D), lambda i, te, tv: (i, 0)),
    )
    return pl.pallas_call(
        _gemm_body,
        grid_spec=grid_spec,
        out_shape=jax.ShapeDtypeStruct((P, D), jnp.float32),
        compiler_params=pltpu.CompilerParams(
            dimension_semantics=("arbitrary",)),
    )(tile_expert, tile_valid, xs, fc_w, proj_w, wrow)


def kernel(x, router_w, fc_w, proj_w):
    b, s, d = x.shape
    xf = x.reshape(s, d)

    logits, sel, rw = _router(xf, router_w.T)

    e = sel.reshape(-1)
    onehot = (e[:, None] == jnp.arange(E, dtype=e.dtype)).astype(jnp.int16)
    cinc = jnp.cumsum(onehot, axis=0)
    counts = cinc[-1].astype(jnp.int32)
    rank = jnp.take_along_axis(cinc, e[:, None], axis=1)[:, 0].astype(jnp.int32) - 1
    tiles_per = (counts + B - 1) // B
    tile_bound = jnp.cumsum(tiles_per)
    row_start = jnp.concatenate(
        [jnp.zeros((1,), jnp.int32), tile_bound[:-1].astype(jnp.int32)]) * B
    pos = row_start[e] + rank

    tok = jnp.arange(NPAIR, dtype=jnp.int32) // 2
    gtok = (jnp.arange(P, dtype=jnp.int32) % S).at[pos].set(tok)
    wrow = jnp.zeros((P, 1), jnp.float32).at[pos, 0].set(rw.reshape(-1))
    tile_ids = jnp.arange(N_TILES, dtype=jnp.int32)
    tile_expert = jnp.minimum(
        jnp.searchsorted(tile_bound, tile_ids, side="right"), E - 1
    ).astype(jnp.int32)
    tile_valid = (tile_ids < tile_bound[-1]).astype(jnp.int32)

    xs = _sc_gather()(xf, gtok)
    ys = _gemm(tile_expert, tile_valid, xs, fc_w, proj_w, wrow)

    posm = pos.reshape(s, 2)
    out = _sc_combine()(ys, posm[:, 0], posm[:, 1])
    return out.reshape(b, s, d), logits


# speedup vs baseline: 1.0886x; 1.0353x over previous
import functools

import jax
import jax.numpy as jnp
from jax import lax
from jax.experimental import pallas as pl
from jax.experimental.pallas import tpu as pltpu
from jax.experimental.pallas import tpu_sc as plsc

D = 768
H = 4 * D
E = 64
S = 4096
NPAIR = 2 * S
B = 256
N_TILES = NPAIR // B + E
P = N_TILES * B

NW = 32
CH = 64


def _router_body(x_ref, wt_ref, logits_ref, sel_ref, rw_ref):
    xb = x_ref[...]
    logits = jnp.dot(xb, wt_ref[...], preferred_element_type=jnp.float32)
    logits_ref[...] = logits
    m = jnp.max(logits, axis=1, keepdims=True)
    p = jnp.exp(logits - m)
    probs = p / jnp.sum(p, axis=1, keepdims=True)
    iota = lax.broadcasted_iota(jnp.int32, probs.shape, 1)
    p1 = jnp.max(probs, axis=1, keepdims=True)
    i1 = jnp.min(jnp.where(probs == p1, iota, E), axis=1, keepdims=True)
    probs2 = jnp.where(iota == i1, -jnp.inf, probs)
    p2 = jnp.max(probs2, axis=1, keepdims=True)
    i2 = jnp.min(jnp.where(probs2 == p2, iota, E), axis=1, keepdims=True)
    s = p1 + p2
    sel_ref[:, 0:1] = i1
    sel_ref[:, 1:2] = i2
    rw_ref[:, 0:1] = p1 / s
    rw_ref[:, 1:2] = p2 / s


def _router(xf, wt):
    return pl.pallas_call(
        _router_body,
        out_shape=(
            jax.ShapeDtypeStruct((S, E), jnp.float32),
            jax.ShapeDtypeStruct((S, 2), jnp.int32),
            jax.ShapeDtypeStruct((S, 2), jnp.float32),
        ),
    )(xf, wt)


def _wid():
    return lax.axis_index("s") * 2 + lax.axis_index("c")


_RPW = P // NW
_NCH = _RPW // CH


def _sc_gather_body(x_hbm, idx_hbm, out_hbm, idx_v, b0, b1, g0, g1, w0, w1):
    base = _wid() * _RPW
    pltpu.sync_copy(idx_hbm.at[pl.ds(base, _RPW)], idx_v)
    bufs, gsem, wsem = (b0, b1), (g0, g1), (w0, w1)
    gcp = [None, None]
    wcp = [None, None]
    gcp[0] = pltpu.async_copy(x_hbm.at[idx_v.at[pl.ds(0, CH)]], bufs[0], gsem[0])
    for i in range(_NCH):
        b = i % 2
        if i + 1 < _NCH:
            nb = (i + 1) % 2
            if wcp[nb] is not None:
                wcp[nb].wait()
            gcp[nb] = pltpu.async_copy(
                x_hbm.at[idx_v.at[pl.ds((i + 1) * CH, CH)]], bufs[nb], gsem[nb])
        gcp[b].wait()
        wcp[b] = pltpu.async_copy(
            bufs[b], out_hbm.at[pl.ds(base + i * CH, CH)], wsem[b])
    wcp[(_NCH - 1) % 2].wait()
    if _NCH > 1:
        wcp[_NCH % 2].wait()


@functools.cache
def _sc_gather():
    return pl.kernel(
        _sc_gather_body,
        out_type=jax.ShapeDtypeStruct((P, D), jnp.float32),
        mesh=plsc.VectorSubcoreMesh(core_axis_name="c", subcore_axis_name="s"),
        scratch_types=[
            pltpu.VMEM((_RPW,), jnp.int32),
            pltpu.VMEM((CH, D), jnp.float32),
            pltpu.VMEM((CH, D), jnp.float32),
            pltpu.SemaphoreType.DMA,
            pltpu.SemaphoreType.DMA,
            pltpu.SemaphoreType.DMA,
            pltpu.SemaphoreType.DMA,
        ],
    )


def _sc_combine_body(ys_hbm, p0_hbm, p1_hbm, out_hbm, i0_v, i1_v, g0_v, g1_v, s0, s1):
    base = _wid() * (S // NW)

    def chunk(ci, carry):
        off = base + ci * CH
        pltpu.sync_copy(p0_hbm.at[pl.ds(off, CH)], i0_v)
        pltpu.sync_copy(p1_hbm.at[pl.ds(off, CH)], i1_v)
        c0 = pltpu.async_copy(ys_hbm.at[i0_v], g0_v, s0)
        c1 = pltpu.async_copy(ys_hbm.at[i1_v], g1_v, s1)
        c0.wait()
        c1.wait()

        def add_row(r, c):
            for u in range(D // 16):
                sl = pl.ds(u * 16, 16)
                g0_v[r, sl] = g0_v[r, sl] + g1_v[r, sl]
            return c

        lax.fori_loop(0, CH, add_row, 0)
        pltpu.sync_copy(g0_v, out_hbm.at[pl.ds(off, CH)])
        return carry

    lax.fori_loop(0, S // NW // CH, chunk, 0)


@functools.cache
def _sc_combine():
    return pl.kernel(
        _sc_combine_body,
        out_type=jax.ShapeDtypeStruct((S, D), jnp.float32),
        mesh=plsc.VectorSubcoreMesh(core_axis_name="c", subcore_axis_name="s"),
        scratch_types=[
            pltpu.VMEM((CH,), jnp.int32),
            pltpu.VMEM((CH,), jnp.int32),
            pltpu.VMEM((CH, D), jnp.float32),
            pltpu.VMEM((CH, D), jnp.float32),
            pltpu.SemaphoreType.DMA,
            pltpu.SemaphoreType.DMA,
        ],
    )


def _gelu(h):
    return 0.5 * h * (1.0 + lax.erf(h * (2.0 ** -0.5)))


def _gemm_body(te_ref, tv_ref, xs_ref, fc_ref, pj_ref, w_ref, ys_ref):
    @pl.when(tv_ref[pl.program_id(0)] == 1)
    def _():
        xb = xs_ref[...]
        h = lax.dot_general(xb, fc_ref[0], (((1,), (1,)), ((), ())),
                            preferred_element_type=jnp.float32)
        h = _gelu(h)
        y = lax.dot_general(h, pj_ref[0], (((1,), (1,)), ((), ())),
                            preferred_element_type=jnp.float32)
        ys_ref[...] = y * w_ref[...]


def _gemm(tile_expert, tile_valid, xs, fc_w, proj_w, wrow):
    grid_spec = pltpu.PrefetchScalarGridSpec(
        num_scalar_prefetch=2,
        grid=(N_TILES,),
        in_specs=[
            pl.BlockSpec((B, D), lambda i, te, tv: (i, 0)),
            pl.BlockSpec((1, H, D), lambda i, te, tv: (te[i], 0, 0)),
            pl.BlockSpec((1, D, H), lambda i, te, tv: (te[i], 0, 0)),
            pl.BlockSpec((B, 1), lambda i, te, tv: (i, 0)),
        ],
        out_specs=pl.BlockSpec((B, D), lambda i, te, tv: (i, 0)),
    )
    return pl.pallas_call(
        _gemm_body,
        grid_spec=grid_spec,
        out_shape=jax.ShapeDtypeStruct((P, D), jnp.float32),
        compiler_params=pltpu.CompilerParams(
            dimension_semantics=("arbitrary",)),
    )(tile_expert, tile_valid, xs, fc_w, proj_w, wrow)


def kernel(x, router_w, fc_w, proj_w):
    b, s, d = x.shape
    xf = x.reshape(s, d)

    logits, sel, rw = _router(xf, router_w.T)

    e = sel.reshape(-1)
    onehot = (e[:, None] == jnp.arange(E, dtype=e.dtype)).astype(jnp.int16)
    cinc = jnp.cumsum(onehot, axis=0)
    counts = cinc[-1].astype(jnp.int32)
    rank = jnp.take_along_axis(cinc, e[:, None], axis=1)[:, 0].astype(jnp.int32) - 1
    tiles_per = (counts + B - 1) // B
    tile_bound = jnp.cumsum(tiles_per)
    row_start = jnp.concatenate(
        [jnp.zeros((1,), jnp.int32), tile_bound[:-1].astype(jnp.int32)]) * B
    pos = row_start[e] + rank

    tok = jnp.arange(NPAIR, dtype=jnp.int32) // 2
    gtok = (jnp.arange(P, dtype=jnp.int32) % S).at[pos].set(tok)
    wrow = jnp.zeros((P, 1), jnp.float32).at[pos, 0].set(rw.reshape(-1))
    tile_ids = jnp.arange(N_TILES, dtype=jnp.int32)
    tile_expert = jnp.minimum(
        jnp.searchsorted(tile_bound, tile_ids, side="right"), E - 1
    ).astype(jnp.int32)
    tile_valid = (tile_ids < tile_bound[-1]).astype(jnp.int32)

    xs = _sc_gather()(xf, gtok)
    ys = _gemm(tile_expert, tile_valid, xs, fc_w, proj_w, wrow)

    posm = pos.reshape(s, 2)
    out = _sc_combine()(ys, posm[:, 0], posm[:, 1])
    return out.reshape(b, s, d), logits
